# baseline (speedup 1.0000x reference)
import functools

import jax
import jax.numpy as jnp
from jax import lax
from jax.experimental import pallas as pl
from jax.experimental.pallas import tpu as pltpu
from jax.experimental.pallas import tpu_sc as plsc

B, S, H, E, TOP_K = 2, 2048, 1024, 8, 2
T = B * S
A = T * TOP_K
BMG = 512
BG = 256
P_CAP = A + E * BG
NB = P_CAP // BG
NBP = 48
NC, NS, L = 2, 16, 16
NW = NC * NS
CA = A // NW
CPW = P_CAP // NW
TW = T // NW
PADW = (P_CAP - A) // NW


def _gate_body(x_ref, wg_ref, bg_ref, eidx_ref, gval_ref, hist_ref):
    xb = x_ref[...]
    logits = lax.dot_general(xb, wg_ref[...], (((1,), (1,)), ((), ())),
                             preferred_element_type=jnp.float32)
    logits = logits + bg_ref[...]
    m = jnp.max(logits, axis=1, keepdims=True)
    ex = jnp.exp(logits - m)
    gates = ex / jnp.sum(ex, axis=1, keepdims=True)
    iota = lax.broadcasted_iota(jnp.int32, (BMG, E), 1)
    g1 = jnp.max(gates, axis=1, keepdims=True)
    i1 = jnp.min(jnp.where(gates == g1, iota, E), axis=1, keepdims=True)
    gates2 = jnp.where(iota == i1, -1.0, gates)
    g2 = jnp.max(gates2, axis=1, keepdims=True)
    i2 = jnp.min(jnp.where(gates2 == g2, iota, E), axis=1, keepdims=True)
    eidx_ref[...] = jnp.concatenate([i1, i2], axis=1)
    gval_ref[...] = jnp.concatenate([g1, g2], axis=1)
    onehot = ((iota == i1) | (iota == i2)).astype(jnp.int32)
    rows = []
    for r in range(BMG // 128):
        h = jnp.sum(onehot[r * 128:(r + 1) * 128], axis=0, keepdims=True)
        rows.append(jnp.concatenate(
            [h, jnp.zeros((1, L - E), jnp.int32)], axis=1))
    hist_ref[...] = jnp.concatenate(rows, axis=0)[None]


def _gating(xf, wg, bg2):
    return pl.pallas_call(
        _gate_body,
        grid=(T // BMG,),
        in_specs=[
            pl.BlockSpec((BMG, H), lambda i: (i, 0)),
            pl.BlockSpec((E, H), lambda i: (0, 0)),
            pl.BlockSpec((1, E), lambda i: (0, 0)),
        ],
        out_specs=[
            pl.BlockSpec((BMG, TOP_K), lambda i: (i, 0)),
            pl.BlockSpec((BMG, TOP_K), lambda i: (i, 0)),
            pl.BlockSpec((1, BMG // 128, L), lambda i: (i, 0, 0)),
        ],
        out_shape=[
            jax.ShapeDtypeStruct((T, TOP_K), jnp.int32),
            jax.ShapeDtypeStruct((T, TOP_K), jnp.float32),
            jax.ShapeDtypeStruct((T // BMG, BMG // 128, L), jnp.int32),
        ],
    )(xf, wg, bg2)


_MESH = plsc.VectorSubcoreMesh(core_axis_name="c", subcore_axis_name="s")


def _lane(vec, iota, e):
    return jnp.sum(jnp.where(iota == e, vec, 0))


@functools.partial(
    pl.kernel,
    out_type=[
        jax.ShapeDtypeStruct((P_CAP,), jnp.int32),
        jax.ShapeDtypeStruct((P_CAP,), jnp.float32),
        jax.ShapeDtypeStruct((A,), jnp.int32),
        jax.ShapeDtypeStruct((NBP,), jnp.int32),
        jax.ShapeDtypeStruct((NBP,), jnp.int32),
    ],
    mesh=_MESH,
    compiler_params=pltpu.CompilerParams(needs_layout_passes=False),
    scratch_types=[
        pltpu.VMEM((NW, L), jnp.int32),
        pltpu.VMEM((CA,), jnp.int32),
        pltpu.VMEM((CA,), jnp.float32),
        pltpu.VMEM((CA,), jnp.int32),
        pltpu.VMEM((2, 128), jnp.int32),
        pltpu.VMEM((2, 128), jnp.int32),
        pltpu.VMEM((2, 128), jnp.float32),
        pltpu.VMEM((1, PADW), jnp.int32),
        pltpu.VMEM((PADW,), jnp.int32),
        pltpu.VMEM((PADW,), jnp.float32),
        pltpu.VMEM((NBP,), jnp.int32),
        pltpu.VMEM((NBP,), jnp.int32),
        pltpu.SemaphoreType.DMA,
    ],
)
def _route(ef_hbm, gf_hbm, hist_hbm,
           tok_hbm, gate_hbm, pos_hbm, blk_hbm, xsb_hbm,
           tbl_v, ef_v, gf_v, pos_v, posr_v, tokr_v, gr_v,
           pidx_v, zvi_v, zvf_v, blk_v, xsb_v, sem):
    w = lax.axis_index("s") * NC + lax.axis_index("c")
    iota = lax.iota(jnp.int32, L)
    pltpu.sync_copy(hist_hbm, tbl_v)
    pltpu.sync_copy(ef_hbm.at[pl.ds(w * CA, CA)], ef_v)
    pltpu.sync_copy(gf_hbm.at[pl.ds(w * CA, CA)], gf_v)
    cnt = jnp.zeros((L,), jnp.int32)
    part = jnp.zeros((L,), jnp.int32)
    for c in range(NW):
        row = tbl_v[c]
        cnt = cnt + row
        part = part + row * jnp.where(w > c, 1, 0)
    used_pad = ((cnt + (BG - 1)) >> 8) << 8
    offp_inc = plsc.cumsum(used_pad)
    offp = offp_inc - used_pad
    start = offp + cnt
    pad_e = used_pad - cnt
    padcum_inc = plsc.cumsum(pad_e)
    padcum_excl = padcum_inc - pad_e
    used_total = jnp.max(offp_inc)
    padcum_total = jnp.max(padcum_inc)
    offp_s = [_lane(offp, iota, e) for e in range(E)]
    start_s = [_lane(start, iota, e) for e in range(E)]
    pexcl_s = [_lane(padcum_excl, iota, e) for e in range(E)]
    pinc_s = [_lane(padcum_inc, iota, e) for e in range(E)]
    base_s = [offp_s[e] + _lane(part, iota, e) for e in range(E)]
    for i in range(CA // L):
        ev = ef_v[pl.ds(i * L, L)]
        gv = gf_v[pl.ds(i * L, L)]
        tokv = (w * CA + i * L + iota) >> 1
        posv = jnp.zeros((L,), jnp.int32)
        for e in range(E):
            mi = jnp.where(ev == e, 1, 0)
            pc = plsc.cumsum(mi)
            posv = jnp.where(ev == e, (pc - mi) + base_s[e], posv)
            base_s[e] = base_s[e] + jnp.max(pc)
        pos_v[pl.ds(i * L, L)] = posv
        posr_v[i // 8, pl.ds((i % 8) * L, L)] = posv
        tokr_v[i // 8, pl.ds((i % 8) * L, L)] = tokv
        gr_v[i // 8, pl.ds((i % 8) * L, L)] = gv
    for r in range(PADW // L):
        jv = w * PADW + r * L + iota
        ecnt = jnp.zeros((L,), jnp.int32)
        for e in range(E):
            ecnt = ecnt + jnp.where(jv >= pinc_s[e], 1, 0)
        slot = used_total + (jv - padcum_total)
        for e in range(E):
            slot = jnp.where(ecnt == e, start_s[e] + (jv - pexcl_s[e]), slot)
        pidx_v[0, pl.ds(r * L, L)] = slot
        zvi_v[pl.ds(r * L, L)] = jnp.zeros((L,), jnp.int32)
        zvf_v[pl.ds(r * L, L)] = jnp.zeros((L,), jnp.float32)
    pltpu.sync_copy(pos_v, pos_hbm.at[pl.ds(w * CA, CA)])
    for c in range(2):
        pltpu.async_copy(tokr_v.at[c], tok_hbm.at[posr_v.at[c]], sem).wait()
        pltpu.async_copy(gr_v.at[c], gate_hbm.at[posr_v.at[c]], sem).wait()
    pltpu.async_copy(zvi_v, tok_hbm.at[pidx_v.at[0]], sem).wait()
    pltpu.async_copy(zvf_v, gate_hbm.at[pidx_v.at[0]], sem).wait()
    @pl.when(w == 0)
    def _():
        for r in range(NBP // L):
            bv = (r * L + iota) * BG
            blk = jnp.full((L,), -1, jnp.int32)
            for e in range(E):
                blk = blk + jnp.where(bv >= offp_s[e], 1, 0)
            blk_v[pl.ds(r * L, L)] = blk
            xsb_v[pl.ds(r * L, L)] = jnp.where(
                bv < used_total, r * L + iota, 0)
        pltpu.sync_copy(blk_v, blk_hbm)
        pltpu.sync_copy(xsb_v, xsb_hbm)


@functools.partial(
    pl.kernel,
    out_type=jax.ShapeDtypeStruct((P_CAP, H), jnp.float32),
    mesh=_MESH,
    compiler_params=pltpu.CompilerParams(needs_layout_passes=False),
    scratch_types=[
        pltpu.VMEM((64,), jnp.int32),
        pltpu.VMEM((64, H), jnp.float32),
        pltpu.SemaphoreType.DMA,
    ],
)
def _gather_rows(tok_hbm, xf_hbm, xs_hbm, idx_v, rows_v, sem):
    w = lax.axis_index("s") * NC + lax.axis_index("c")
    for c in range(CPW // 64):
        off = w * CPW + c * 64
        pltpu.sync_copy(tok_hbm.at[pl.ds(off, 64)], idx_v)
        pltpu.async_copy(xf_hbm.at[idx_v], rows_v, sem).wait()
        pltpu.sync_copy(rows_v, xs_hbm.at[pl.ds(off, 64)])


def _gmm_body(bexp_ref, xsb_ref, xs_ref, wet_ref, be_ref, g_ref, ys_ref):
    xb = xs_ref[...].astype(jnp.bfloat16)
    mm = jnp.dot(xb, wet_ref[0], preferred_element_type=jnp.float32)
    g = g_ref[...]
    ys_ref[...] = g * mm + g * be_ref[0]


def _gmm(blk_expert, xs_blk, xs, wet16, be, gate2):
    return pl.pallas_call(
        _gmm_body,
        grid_spec=pltpu.PrefetchScalarGridSpec(
            num_scalar_prefetch=2,
            grid=(NB,),
            in_specs=[
                pl.BlockSpec((BG, H), lambda b, bexp, xsb: (xsb[b], 0)),
                pl.BlockSpec((1, H, H), lambda b, bexp, xsb: (bexp[b], 0, 0)),
                pl.BlockSpec((1, 1, H), lambda b, bexp, xsb: (bexp[b], 0, 0)),
                pl.BlockSpec((BG, 1), lambda b, bexp, xsb: (xsb[b], 0)),
            ],
            out_specs=pl.BlockSpec((BG, H), lambda b, bexp, xsb: (b, 0)),
        ),
        out_shape=jax.ShapeDtypeStruct((P_CAP, H), jnp.float32),
    )(blk_expert, xs_blk, xs, wet16, be.reshape(E, 1, H), gate2)


@functools.partial(
    pl.kernel,
    out_type=jax.ShapeDtypeStruct((T, H), jnp.float32),
    mesh=_MESH,
    compiler_params=pltpu.CompilerParams(needs_layout_passes=False),
    scratch_types=[
        pltpu.VMEM((64,), jnp.int32),
        pltpu.VMEM((64, H), jnp.float32),
        pltpu.VMEM((32, H), jnp.float32),
        pltpu.SemaphoreType.DMA,
    ],
)
def _combine(ys_hbm, pos_hbm, out_hbm, pidx_v, rows_v, out_v, sem):
    w = lax.axis_index("s") * NC + lax.axis_index("c")
    for c in range(TW // 32):
        t0 = w * TW + c * 32
        pltpu.sync_copy(pos_hbm.at[pl.ds(2 * t0, 64)], pidx_v)
        pltpu.async_copy(ys_hbm.at[pidx_v], rows_v, sem).wait()

        @plsc.parallel_loop(0, 32 * (H // L), 1, unroll=8)
        def _(i):
            j = lax.shift_right_logical(i, 6)
            l16 = jnp.bitwise_and(i, (H // L) - 1) * L
            out_v[j, pl.ds(l16, L)] = (
                rows_v[2 * j, pl.ds(l16, L)]
                + rows_v[2 * j + 1, pl.ds(l16, L)])

        pltpu.sync_copy(out_v, out_hbm.at[pl.ds(t0, 32)])


_SC_ROUTE, _SC_GATHER, _SC_COMBINE = True, True, True


def _route_jnp(ef, gf, hist):
    cnt = jnp.sum(hist, axis=0)[:E]
    used_pad = ((cnt + BG - 1) // BG) * BG
    offp = jnp.concatenate([jnp.zeros((1,), jnp.int32),
                            jnp.cumsum(used_pad)[:-1].astype(jnp.int32)])
    order = jnp.argsort(ef, stable=True)
    sorted_e = ef[order]
    start = jnp.concatenate([jnp.zeros((1,), jnp.int32),
                             jnp.cumsum(cnt)[:-1].astype(jnp.int32)])
    posn = offp[sorted_e] + (jnp.arange(A, dtype=jnp.int32) - start[sorted_e])
    pos = jnp.zeros((A,), jnp.int32).at[order].set(posn)
    tok_sorted = jnp.zeros((P_CAP,), jnp.int32).at[pos].set(
        jnp.arange(A, dtype=jnp.int32) // TOP_K)
    gate_sorted = jnp.zeros((P_CAP,), jnp.float32).at[pos].set(gf)
    used_total = jnp.sum(used_pad)
    bv = jnp.arange(NBP, dtype=jnp.int32)
    blk_e = (jnp.sum(offp[None, :] <= (bv * BG)[:, None], axis=1) - 1
             ).astype(jnp.int32)
    xs_blk = jnp.where(bv * BG < used_total, bv, 0).astype(jnp.int32)
    return tok_sorted, gate_sorted, pos, blk_e, xs_blk


@jax.jit
def _moe(xf, wg, bg2, wet16, be):
    eidx, gval, hist = _gating(xf, wg, bg2)
    ef = eidx.reshape(A)
    gf = gval.reshape(A)
    if _SC_ROUTE:
        tok_sorted, gate_sorted, pos, blk_e, xs_blk = _route(
            ef, gf, hist.reshape(NW, L))
    else:
        tok_sorted, gate_sorted, pos, blk_e, xs_blk = _route_jnp(
            ef, gf, hist.reshape(NW, L))
    if _SC_GATHER:
        xs = _gather_rows(tok_sorted, xf)
    else:
        xs = xf[tok_sorted]
    ys = _gmm(blk_e, xs_blk, xs, wet16, be, gate_sorted.reshape(P_CAP, 1))
    if _SC_COMBINE:
        return _combine(ys, pos)
    pos2 = pos.reshape(T, TOP_K)
    return ys[pos2[:, 0]] + ys[pos2[:, 1]]


def kernel(x, Wg, bg, We, be):
    xf = x.reshape(T, H)
    wet16 = We.transpose(0, 2, 1).astype(jnp.bfloat16)
    out = _moe(xf, Wg, bg.reshape(1, E), wet16, be)
    return out.reshape(B, S, H)


# speedup vs baseline: 2.6225x; 2.6225x over previous
import jax
import jax.numpy as jnp
from jax import lax
from jax.experimental import pallas as pl
from jax.experimental.pallas import tpu as pltpu

B, S, H, E, TOP_K = 2, 2048, 1024, 8, 2
T = B * S
BM = 512


def _moe_body(x_ref, wg_ref, bg_ref, wall_ref, be_ref, out_ref):
    xb = x_ref[...]
    logits = lax.dot_general(xb, wg_ref[...], (((1,), (1,)), ((), ())),
                             preferred_element_type=jnp.float32)
    logits = logits + bg_ref[...]
    m = jnp.max(logits, axis=1, keepdims=True)
    ex = jnp.exp(logits - m)
    gates = ex / jnp.sum(ex, axis=1, keepdims=True)
    g1 = jnp.max(gates, axis=1, keepdims=True)
    gates_no1 = jnp.where(gates == g1, -1.0, gates)
    g2 = jnp.max(gates_no1, axis=1, keepdims=True)
    w = jnp.where(gates >= g2, gates, 0.0)
    w16 = w.astype(jnp.bfloat16)
    xb16 = xb.astype(jnp.bfloat16)
    lhs = jnp.concatenate(
        [xb16 * w16[:, e:e + 1] for e in range(E)], axis=1)
    mm = jnp.dot(lhs, wall_ref[...], preferred_element_type=jnp.float32)
    out_ref[...] = mm + jnp.dot(w, be_ref[...],
                                preferred_element_type=jnp.float32)


@jax.jit
def _moe(xf, wg, bg2, wall, be):
    return pl.pallas_call(
        _moe_body,
        grid=(T // BM,),
        in_specs=[
            pl.BlockSpec((BM, H), lambda i: (i, 0)),
            pl.BlockSpec((E, H), lambda i: (0, 0)),
            pl.BlockSpec((1, E), lambda i: (0, 0)),
            pl.BlockSpec((E * H, H), lambda i: (0, 0)),
            pl.BlockSpec((E, H), lambda i: (0, 0)),
        ],
        out_specs=pl.BlockSpec((BM, H), lambda i: (i, 0)),
        out_shape=jax.ShapeDtypeStruct((T, H), jnp.float32),
        compiler_params=pltpu.CompilerParams(
            dimension_semantics=("parallel",),
        ),
    )(xf, wg, bg2, wall, be)


def kernel(x, Wg, bg, We, be):
    xf = x.reshape(T, H)
    wall = We.transpose(0, 2, 1).reshape(E * H, H).astype(jnp.bfloat16)
    out = _moe(xf, Wg, bg.reshape(1, E), wall, be)
    return out.reshape(B, S, H)
